# baseline (speedup 1.0000x reference)
import functools

import jax
import jax.numpy as jnp
from jax import lax
from jax.experimental import pallas as pl
from jax.experimental.pallas import tpu as pltpu
from jax.experimental.pallas import tpu_sc as plsc

NC = 2
NS = 16
NW = NC * NS
L = 16
U = 1


def _round_up(a, b):
  return (a + b - 1) // b * b


def _sc_mesh():
  return plsc.VectorSubcoreMesh(core_axis_name="c", subcore_axis_name="s")


def _make_deg_kernel(n_pad, rpt, nchunk, ch, d):
  ng = nchunk
  nb = (ng - 2) // 4

  @functools.partial(
      pl.kernel,
      out_type=(
          jax.ShapeDtypeStruct((NC, n_pad, d), jnp.float32),
          jax.ShapeDtypeStruct((NC, n_pad, d), jnp.float32),
      ),
      mesh=_sc_mesh(),
      scratch_types=(
          [pltpu.VMEM((ch,), jnp.int32)] * 4
          + [pltpu.VMEM((ch, d), jnp.float32)]
          + [pltpu.VMEM_SHARED((n_pad, d), jnp.float32)]
          + [pltpu.SemaphoreType.DMA] * 5
      ),
  )
  def deg_kernel(src_hbm, dst_hbm, ones_hbm, zrow_hbm, odeg_hbm, ideg_hbm,
                 *scr):
    idx = scr[0:4]
    ones_v = scr[4]
    acc_sp = scr[5]
    isms = scr[6:10]
    ssem = scr[10]

    c = lax.axis_index("c")
    s = lax.axis_index("s")
    wid = s * NC + c
    ew = nchunk * ch

    def one_pass(e_hbm):
      def fire_idx(g, k):
        off = wid * ew + g * ch
        pltpu.async_copy(e_hbm.at[pl.ds(off, ch)], idx[k], isms[k])

      def drain_idx(g, k):
        off = wid * ew + g * ch
        pltpu.make_async_copy(e_hbm.at[pl.ds(off, ch)], idx[k],
                              isms[k]).wait()

      def drain_sc(k):
        pltpu.make_async_copy(ones_v, acc_sp.at[idx[k]], ssem).wait()

      def run(g, k):
        drain_idx(g, k)
        pltpu.async_copy(ones_v, acc_sp.at[idx[k]], ssem, add=True)

      fire_idx(0, 0)
      fire_idx(1, 1)

      def body(j, carry):
        g = 4 * j

        @pl.when(j >= 1)
        def _():
          drain_sc(2)
        fire_idx(g + 2, 2)
        run(g, 0)

        @pl.when(j >= 1)
        def _():
          drain_sc(3)
        fire_idx(g + 3, 3)
        run(g + 1, 1)

        drain_sc(0)
        fire_idx(g + 4, 0)
        run(g + 2, 2)

        drain_sc(1)
        fire_idx(g + 5, 1)
        run(g + 3, 3)
        return carry

      lax.fori_loop(0, nb, body, 0)
      for g in range(4 * nb, ng):
        if g >= 2:
          drain_sc((g - 2) % 4)
        if g + 2 < ng:
          fire_idx(g + 2, (g + 2) % 4)
        run(g, g % 4)
      drain_sc((ng - 2) % 4)
      drain_sc((ng - 1) % 4)

    pltpu.sync_copy(zrow_hbm, acc_sp.at[pl.ds(s * rpt, rpt)])
    pltpu.sync_copy(ones_hbm, ones_v)
    plsc.subcore_barrier()
    one_pass(src_hbm)
    plsc.subcore_barrier()
    pltpu.sync_copy(acc_sp.at[pl.ds(s * rpt, rpt)],
                    odeg_hbm.at[c, pl.ds(s * rpt, rpt)])
    pltpu.sync_copy(zrow_hbm, acc_sp.at[pl.ds(s * rpt, rpt)])
    plsc.subcore_barrier()
    one_pass(dst_hbm)
    plsc.subcore_barrier()
    pltpu.sync_copy(acc_sp.at[pl.ds(s * rpt, rpt)],
                    ideg_hbm.at[c, pl.ds(s * rpt, rpt)])

  return deg_kernel


def _make_edge_kernel(n_pad, rpt, nchunk, ch, d):
  ng = nchunk // U
  nb = (ng - 2) // 4

  @functools.partial(
      pl.kernel,
      out_type=jax.ShapeDtypeStruct((NC, n_pad, d), jnp.float32),
      mesh=_sc_mesh(),
      scratch_types=(
          [pltpu.VMEM((ch,), jnp.int32)] * (8 * U)
          + [pltpu.VMEM((ch, d), jnp.float32)] * (2 * U)
          + [pltpu.VMEM_SHARED((n_pad, d), jnp.float32)]
          + [pltpu.SemaphoreType.DMA] * 6
      ),
  )
  def edge_kernel(h_hbm, src_hbm, dst_hbm, zrows_hbm, out_hbm, *scr):
    idx_sets = [(scr[2 * k * U:(2 * k + 1) * U],
                 scr[(2 * k + 1) * U:(2 * k + 2) * U]) for k in range(4)]
    rows_sets = [scr[8 * U:9 * U], scr[9 * U:10 * U]]
    agg_sp = scr[10 * U]
    isms = scr[10 * U + 1:10 * U + 5]
    gsem, ssem = scr[10 * U + 5], scr[10 * U + 6]

    c = lax.axis_index("c")
    s = lax.axis_index("s")
    wid = s * NC + c
    ew = nchunk * ch

    pltpu.sync_copy(zrows_hbm, agg_sp.at[pl.ds(s * rpt, rpt)])
    plsc.subcore_barrier()

    def fire_idx(g, k):
      idxs, idxd = idx_sets[k]
      for b in range(U):
        off = wid * ew + (g * U + b) * ch
        pltpu.async_copy(src_hbm.at[pl.ds(off, ch)], idxs[b], isms[k])
        pltpu.async_copy(dst_hbm.at[pl.ds(off, ch)], idxd[b], isms[k])

    def drain_idx(g, k):
      idxs, idxd = idx_sets[k]
      for b in range(U):
        off = wid * ew + (g * U + b) * ch
        pltpu.make_async_copy(src_hbm.at[pl.ds(off, ch)], idxs[b],
                              isms[k]).wait()
        pltpu.make_async_copy(dst_hbm.at[pl.ds(off, ch)], idxd[b],
                              isms[k]).wait()

    def drain_scatters(k, r):
      idxd = idx_sets[k][1]
      rows = rows_sets[r]
      for b in range(U):
        pltpu.make_async_copy(rows[b], agg_sp.at[idxd[b]], ssem).wait()

    def run_group(g, k, r):
      idxs, idxd = idx_sets[k]
      rows = rows_sets[r]
      drain_idx(g, k)
      descs = [
          pltpu.async_copy(h_hbm.at[idxs[b]], rows[b], gsem) for b in range(U)
      ]
      for desc in descs:
        desc.wait()
      for b in range(U):
        pltpu.async_copy(rows[b], agg_sp.at[idxd[b]], ssem, add=True)

    fire_idx(0, 0)
    fire_idx(1, 1)

    def body(j, carry):
      g = 4 * j

      @pl.when(j >= 1)
      def _():
        drain_scatters(2, 0)
      fire_idx(g + 2, 2)
      run_group(g, 0, 0)

      @pl.when(j >= 1)
      def _():
        drain_scatters(3, 1)
      fire_idx(g + 3, 3)
      run_group(g + 1, 1, 1)

      drain_scatters(0, 0)
      fire_idx(g + 4, 0)
      run_group(g + 2, 2, 0)

      drain_scatters(1, 1)
      fire_idx(g + 5, 1)
      run_group(g + 3, 3, 1)
      return carry

    lax.fori_loop(0, nb, body, 0)
    for g in range(4 * nb, ng):
      if g >= 2:
        drain_scatters((g - 2) % 4, (g - 2) % 2)
      if g + 2 < ng:
        fire_idx(g + 2, (g + 2) % 4)
      run_group(g, g % 4, g % 2)
    drain_scatters((ng - 2) % 4, (ng - 2) % 2)
    drain_scatters((ng - 1) % 4, (ng - 1) % 2)

    plsc.subcore_barrier()
    pltpu.sync_copy(agg_sp.at[pl.ds(s * rpt, rpt)],
                    out_hbm.at[c, pl.ds(s * rpt, rpt)])

  return edge_kernel


def _tc_norms_and_first_matmul(x, w1, odeg_p, ideg_p, blk):
  n, d = x.shape

  def body(x_ref, w_ref, od_ref, id_ref, h_ref, ns_ref, nd_ref):
    od = od_ref[0, :, 0:1] + od_ref[1, :, 0:1]
    idg = id_ref[0, :, 0:1] + id_ref[1, :, 0:1]
    ns = lax.rsqrt(jnp.maximum(od, 1.0))
    nd = lax.rsqrt(jnp.maximum(idg, 1.0))
    h = jnp.dot(x_ref[...], w_ref[...], preferred_element_type=jnp.float32)
    h_ref[...] = h * ns
    ns_ref[...] = ns
    nd_ref[...] = nd

  return pl.pallas_call(
      body,
      grid=(n // blk,),
      in_specs=[
          pl.BlockSpec((blk, d), lambda i: (i, 0)),
          pl.BlockSpec((d, d), lambda i: (0, 0)),
          pl.BlockSpec((NC, blk, d), lambda i: (0, i, 0)),
          pl.BlockSpec((NC, blk, d), lambda i: (0, i, 0)),
      ],
      out_specs=[
          pl.BlockSpec((blk, d), lambda i: (i, 0)),
          pl.BlockSpec((blk, 1), lambda i: (i, 0)),
          pl.BlockSpec((blk, 1), lambda i: (i, 0)),
      ],
      out_shape=[
          jax.ShapeDtypeStruct((n, d), jnp.float32),
          jax.ShapeDtypeStruct((n, 1), jnp.float32),
          jax.ShapeDtypeStruct((n, 1), jnp.float32),
      ],
  )(x, w1, odeg_p, ideg_p)


def _tc_mid(parts, nd, b, ns, w2, n, blk):
  d = parts.shape[-1]

  def body(p_ref, nd_ref, b_ref, ns_ref, w_ref, o_ref):
    agg = p_ref[0] + p_ref[1]
    h1 = jnp.maximum(agg * nd_ref[...] + b_ref[...], 0.0)
    h = jnp.dot(h1, w_ref[...], preferred_element_type=jnp.float32)
    o_ref[...] = h * ns_ref[...]

  return pl.pallas_call(
      body,
      grid=(n // blk,),
      in_specs=[
          pl.BlockSpec((NC, blk, d), lambda i: (0, i, 0)),
          pl.BlockSpec((blk, 1), lambda i: (i, 0)),
          pl.BlockSpec((1, d), lambda i: (0, 0)),
          pl.BlockSpec((blk, 1), lambda i: (i, 0)),
          pl.BlockSpec((d, d), lambda i: (0, 0)),
      ],
      out_specs=pl.BlockSpec((blk, d), lambda i: (i, 0)),
      out_shape=jax.ShapeDtypeStruct((n, d), jnp.float32),
  )(parts, nd, b, ns, w2)


def _tc_final(parts, nd, b, n, blk):
  d = parts.shape[-1]

  def body(p_ref, nd_ref, b_ref, o_ref):
    agg = p_ref[0] + p_ref[1]
    o_ref[...] = jnp.maximum(agg * nd_ref[...] + b_ref[...], 0.0)

  return pl.pallas_call(
      body,
      grid=(n // blk,),
      in_specs=[
          pl.BlockSpec((NC, blk, d), lambda i: (0, i, 0)),
          pl.BlockSpec((blk, 1), lambda i: (i, 0)),
          pl.BlockSpec((1, d), lambda i: (0, 0)),
      ],
      out_specs=pl.BlockSpec((blk, d), lambda i: (i, 0)),
      out_shape=jax.ShapeDtypeStruct((n, d), jnp.float32),
  )(parts, nd, b)


def kernel(feats, edge_index, W1, b1, W2, b2):
  n, d = feats.shape
  e = edge_index.shape[1]
  ew = e // NW
  ch_e = 128
  ew_p = _round_up(ew, ch_e)
  nchunk_e = ew_p // ch_e
  assert nchunk_e % U == 0 and nchunk_e // U >= 2
  rpt = _round_up(-(-n // NS), 8)
  n_pad = rpt * NS
  blk = 1000

  pad = ((0, 0), (0, ew_p - ew))
  src_g = jnp.pad(edge_index[0].reshape(NW, ew), pad).reshape(-1)
  dst_r = jnp.pad(edge_index[1].reshape(NW, ew), pad,
                  constant_values=n_pad - 1).reshape(-1)
  src_d = jnp.pad(edge_index[0].reshape(NW, ew), pad,
                  constant_values=n_pad - 1).reshape(-1)
  ones_rows = jnp.ones((ch_e, d), jnp.float32)
  zrow_feat = jnp.zeros((rpt, d), jnp.float32)

  deg_k = _make_deg_kernel(n_pad, rpt, nchunk_e, ch_e, d)
  odeg_p, ideg_p = deg_k(src_d, dst_r, ones_rows, zrow_feat)

  h1pre, ns, nd = _tc_norms_and_first_matmul(feats, W1, odeg_p, ideg_p, blk)

  edge_k = _make_edge_kernel(n_pad, rpt, nchunk_e, ch_e, d)
  p1 = edge_k(h1pre, src_g, dst_r, zrow_feat)
  h2pre = _tc_mid(p1, nd, b1.reshape(1, d), ns, W2, n, blk)
  p2 = edge_k(h2pre, src_g, dst_r, zrow_feat)
  return _tc_final(p2, nd, b2.reshape(1, d), n, blk)


# speedup vs baseline: 1.3926x; 1.3926x over previous
import functools

import jax
import jax.numpy as jnp
from jax import lax
from jax.experimental import pallas as pl
from jax.experimental.pallas import tpu as pltpu
from jax.experimental.pallas import tpu_sc as plsc

NC = 2
NS = 16
NW = NC * NS
L = 16
U = 1


def _round_up(a, b):
  return (a + b - 1) // b * b


def _sc_mesh():
  return plsc.VectorSubcoreMesh(core_axis_name="c", subcore_axis_name="s")


def _make_deg_kernel(n_pad, rpt, nchunk, ch, d):
  ng = nchunk
  nb = (ng - 2) // 4

  @functools.partial(
      pl.kernel,
      out_type=(
          jax.ShapeDtypeStruct((NC, n_pad, d), jnp.float32),
          jax.ShapeDtypeStruct((NC, n_pad, d), jnp.float32),
      ),
      mesh=_sc_mesh(),
      scratch_types=(
          [pltpu.VMEM((ch,), jnp.int32)] * 4
          + [pltpu.VMEM((ch, d), jnp.float32)]
          + [pltpu.VMEM_SHARED((n_pad, d), jnp.float32)]
          + [pltpu.SemaphoreType.DMA] * 5
      ),
  )
  def deg_kernel(src_hbm, dst_hbm, ones_hbm, zrow_hbm, odeg_hbm, ideg_hbm,
                 *scr):
    idx = scr[0:4]
    ones_v = scr[4]
    acc_sp = scr[5]
    isms = scr[6:10]
    ssem = scr[10]

    c = lax.axis_index("c")
    s = lax.axis_index("s")
    wid = s * NC + c
    ew = nchunk * ch

    def one_pass(e_hbm):
      def fire_idx(g, k):
        off = wid * ew + g * ch
        pltpu.async_copy(e_hbm.at[pl.ds(off, ch)], idx[k], isms[k])

      def drain_idx(g, k):
        off = wid * ew + g * ch
        pltpu.make_async_copy(e_hbm.at[pl.ds(off, ch)], idx[k],
                              isms[k]).wait()

      def drain_sc(k):
        pltpu.make_async_copy(ones_v, acc_sp.at[idx[k]], ssem).wait()

      def run(g, k):
        drain_idx(g, k)
        pltpu.async_copy(ones_v, acc_sp.at[idx[k]], ssem, add=True)

      fire_idx(0, 0)
      fire_idx(1, 1)

      def body(j, carry):
        g = 4 * j

        @pl.when(j >= 1)
        def _():
          drain_sc(2)
        fire_idx(g + 2, 2)
        run(g, 0)

        @pl.when(j >= 1)
        def _():
          drain_sc(3)
        fire_idx(g + 3, 3)
        run(g + 1, 1)

        drain_sc(0)
        fire_idx(g + 4, 0)
        run(g + 2, 2)

        drain_sc(1)
        fire_idx(g + 5, 1)
        run(g + 3, 3)
        return carry

      lax.fori_loop(0, nb, body, 0)
      for g in range(4 * nb, ng):
        if g >= 2:
          drain_sc((g - 2) % 4)
        if g + 2 < ng:
          fire_idx(g + 2, (g + 2) % 4)
        run(g, g % 4)
      drain_sc((ng - 2) % 4)
      drain_sc((ng - 1) % 4)

    pltpu.sync_copy(zrow_hbm, acc_sp.at[pl.ds(s * rpt, rpt)])
    pltpu.sync_copy(ones_hbm, ones_v)
    plsc.subcore_barrier()
    one_pass(src_hbm)
    plsc.subcore_barrier()
    pltpu.sync_copy(acc_sp.at[pl.ds(s * rpt, rpt)],
                    odeg_hbm.at[c, pl.ds(s * rpt, rpt)])
    pltpu.sync_copy(zrow_hbm, acc_sp.at[pl.ds(s * rpt, rpt)])
    plsc.subcore_barrier()
    one_pass(dst_hbm)
    plsc.subcore_barrier()
    pltpu.sync_copy(acc_sp.at[pl.ds(s * rpt, rpt)],
                    ideg_hbm.at[c, pl.ds(s * rpt, rpt)])

  return deg_kernel


def _make_edge_kernel(n_pad, rpt, nchunk, ch, d):
  ng = nchunk // U
  nb = (ng - 2) // 4

  @functools.partial(
      pl.kernel,
      out_type=jax.ShapeDtypeStruct((NC, n_pad, d), jnp.float32),
      mesh=_sc_mesh(),
      scratch_types=(
          [pltpu.VMEM((ch,), jnp.int32)] * (8 * U)
          + [pltpu.VMEM((ch, d), jnp.float32)] * (2 * U)
          + [pltpu.VMEM_SHARED((n_pad, d), jnp.float32)]
          + [pltpu.SemaphoreType.DMA] * 6
      ),
  )
  def edge_kernel(h_hbm, src_hbm, dst_hbm, zrows_hbm, out_hbm, *scr):
    idx_sets = [(scr[2 * k * U:(2 * k + 1) * U],
                 scr[(2 * k + 1) * U:(2 * k + 2) * U]) for k in range(4)]
    rows_sets = [scr[8 * U:9 * U], scr[9 * U:10 * U]]
    agg_sp = scr[10 * U]
    isms = scr[10 * U + 1:10 * U + 5]
    gsem, ssem = scr[10 * U + 5], scr[10 * U + 6]

    c = lax.axis_index("c")
    s = lax.axis_index("s")
    wid = s * NC + c
    ew = nchunk * ch

    pltpu.sync_copy(zrows_hbm, agg_sp.at[pl.ds(s * rpt, rpt)])
    plsc.subcore_barrier()

    def fire_idx(g, k):
      idxs, idxd = idx_sets[k]
      for b in range(U):
        off = wid * ew + (g * U + b) * ch
        pltpu.async_copy(src_hbm.at[pl.ds(off, ch)], idxs[b], isms[k])
        pltpu.async_copy(dst_hbm.at[pl.ds(off, ch)], idxd[b], isms[k])

    def drain_idx(g, k):
      idxs, idxd = idx_sets[k]
      for b in range(U):
        off = wid * ew + (g * U + b) * ch
        pltpu.make_async_copy(src_hbm.at[pl.ds(off, ch)], idxs[b],
                              isms[k]).wait()
        pltpu.make_async_copy(dst_hbm.at[pl.ds(off, ch)], idxd[b],
                              isms[k]).wait()

    def drain_scatters(k, r):
      idxd = idx_sets[k][1]
      rows = rows_sets[r]
      for b in range(U):
        pltpu.make_async_copy(rows[b], agg_sp.at[idxd[b]], ssem).wait()

    def run_group(g, k, r):
      idxs, idxd = idx_sets[k]
      rows = rows_sets[r]
      drain_idx(g, k)
      descs = [
          pltpu.async_copy(h_hbm.at[idxs[b]], rows[b], gsem) for b in range(U)
      ]
      for desc in descs:
        desc.wait()
      for b in range(U):
        pltpu.async_copy(rows[b], agg_sp.at[idxd[b]], ssem, add=True)

    fire_idx(0, 0)
    fire_idx(1, 1)

    def body(j, carry):
      g = 4 * j

      @pl.when(j >= 1)
      def _():
        drain_scatters(2, 0)
      fire_idx(g + 2, 2)
      run_group(g, 0, 0)

      @pl.when(j >= 1)
      def _():
        drain_scatters(3, 1)
      fire_idx(g + 3, 3)
      run_group(g + 1, 1, 1)

      drain_scatters(0, 0)
      fire_idx(g + 4, 0)
      run_group(g + 2, 2, 0)

      drain_scatters(1, 1)
      fire_idx(g + 5, 1)
      run_group(g + 3, 3, 1)
      return carry

    lax.fori_loop(0, nb, body, 0)
    for g in range(4 * nb, ng):
      if g >= 2:
        drain_scatters((g - 2) % 4, (g - 2) % 2)
      if g + 2 < ng:
        fire_idx(g + 2, (g + 2) % 4)
      run_group(g, g % 4, g % 2)
    drain_scatters((ng - 2) % 4, (ng - 2) % 2)
    drain_scatters((ng - 1) % 4, (ng - 1) % 2)

    plsc.subcore_barrier()
    pltpu.sync_copy(agg_sp.at[pl.ds(s * rpt, rpt)],
                    out_hbm.at[c, pl.ds(s * rpt, rpt)])

  return edge_kernel


def _tc_norms_and_first_matmul(x, w1, odeg_p, ideg_p, blk):
  n, d = x.shape

  def body(x_ref, w_ref, od_ref, id_ref, h_ref, ns_ref, nd_ref):
    od = od_ref[0, :, 0:1] + od_ref[1, :, 0:1]
    idg = id_ref[0, :, 0:1] + id_ref[1, :, 0:1]
    ns = lax.rsqrt(jnp.maximum(od, 1.0))
    nd = lax.rsqrt(jnp.maximum(idg, 1.0))
    h = jnp.dot(x_ref[...], w_ref[...], preferred_element_type=jnp.float32)
    h_ref[...] = h * ns
    ns_ref[...] = ns
    nd_ref[...] = nd

  return pl.pallas_call(
      body,
      grid=(n // blk,),
      in_specs=[
          pl.BlockSpec((blk, d), lambda i: (i, 0)),
          pl.BlockSpec((d, d), lambda i: (0, 0)),
          pl.BlockSpec((NC, blk, d), lambda i: (0, i, 0)),
          pl.BlockSpec((NC, blk, d), lambda i: (0, i, 0)),
      ],
      out_specs=[
          pl.BlockSpec((blk, d), lambda i: (i, 0)),
          pl.BlockSpec((blk, 1), lambda i: (i, 0)),
          pl.BlockSpec((blk, 1), lambda i: (i, 0)),
      ],
      out_shape=[
          jax.ShapeDtypeStruct((n, d), jnp.float32),
          jax.ShapeDtypeStruct((n, 1), jnp.float32),
          jax.ShapeDtypeStruct((n, 1), jnp.float32),
      ],
  )(x, w1, odeg_p, ideg_p)


def _tc_mid(parts, nd, b, ns, w2, n, blk):
  d = parts.shape[-1]

  def body(p_ref, nd_ref, b_ref, ns_ref, w_ref, o_ref):
    agg = p_ref[0] + p_ref[1]
    h1 = jnp.maximum(agg * nd_ref[...] + b_ref[...], 0.0)
    h = jnp.dot(h1, w_ref[...], preferred_element_type=jnp.float32)
    o_ref[...] = h * ns_ref[...]

  return pl.pallas_call(
      body,
      grid=(n // blk,),
      in_specs=[
          pl.BlockSpec((NC, blk, d), lambda i: (0, i, 0)),
          pl.BlockSpec((blk, 1), lambda i: (i, 0)),
          pl.BlockSpec((1, d), lambda i: (0, 0)),
          pl.BlockSpec((blk, 1), lambda i: (i, 0)),
          pl.BlockSpec((d, d), lambda i: (0, 0)),
      ],
      out_specs=pl.BlockSpec((blk, d), lambda i: (i, 0)),
      out_shape=jax.ShapeDtypeStruct((n, d), jnp.float32),
  )(parts, nd, b, ns, w2)


def _tc_final(parts, nd, b, n, blk):
  d = parts.shape[-1]

  def body(p_ref, nd_ref, b_ref, o_ref):
    agg = p_ref[0] + p_ref[1]
    o_ref[...] = jnp.maximum(agg * nd_ref[...] + b_ref[...], 0.0)

  return pl.pallas_call(
      body,
      grid=(n // blk,),
      in_specs=[
          pl.BlockSpec((NC, blk, d), lambda i: (0, i, 0)),
          pl.BlockSpec((blk, 1), lambda i: (i, 0)),
          pl.BlockSpec((1, d), lambda i: (0, 0)),
      ],
      out_specs=pl.BlockSpec((blk, d), lambda i: (i, 0)),
      out_shape=jax.ShapeDtypeStruct((n, d), jnp.float32),
  )(parts, nd, b)


def kernel(feats, edge_index, W1, b1, W2, b2):
  n, d = feats.shape
  e = edge_index.shape[1]
  ew = e // NW
  ch_e = 80
  ew_p = _round_up(ew, ch_e)
  nchunk_e = ew_p // ch_e
  assert nchunk_e % U == 0 and nchunk_e // U >= 2
  rpt = _round_up(-(-n // NS), 8)
  n_pad = rpt * NS
  blk = 1000

  pad = ((0, 0), (0, ew_p - ew))
  src_g = jnp.pad(edge_index[0].reshape(NW, ew), pad).reshape(-1)
  dst_r = jnp.pad(edge_index[1].reshape(NW, ew), pad,
                  constant_values=n_pad - 1).reshape(-1)
  src_d = jnp.pad(edge_index[0].reshape(NW, ew), pad,
                  constant_values=n_pad - 1).reshape(-1)
  ones_rows = jnp.ones((ch_e, d), jnp.float32)
  zrow_feat = jnp.zeros((rpt, d), jnp.float32)

  deg_k = _make_deg_kernel(n_pad, rpt, nchunk_e, ch_e, d)
  odeg_p, ideg_p = deg_k(src_d, dst_r, ones_rows, zrow_feat)

  h1pre, ns, nd = _tc_norms_and_first_matmul(feats, W1, odeg_p, ideg_p, blk)

  edge_k = _make_edge_kernel(n_pad, rpt, nchunk_e, ch_e, d)
  p1 = edge_k(h1pre, src_g, dst_r, zrow_feat)
  h2pre = _tc_mid(p1, nd, b1.reshape(1, d), ns, W2, n, blk)
  p2 = edge_k(h2pre, src_g, dst_r, zrow_feat)
  return _tc_final(p2, nd, b2.reshape(1, d), n, blk)
